# baseline (speedup 1.0000x reference)
import functools

import jax
import jax.numpy as jnp
from jax import lax
from jax.experimental import pallas as pl
from jax.experimental.pallas import tpu as pltpu
from jax.experimental.pallas import tpu_sc as plsc

EMB_D = 32
ROW_W = EMB_D + 1
LANES = 16


@functools.lru_cache(maxsize=None)
def _sc_geometry():
    try:
        info = plsc.get_sparse_core_info()
        return int(info.num_cores), int(info.num_subcores)
    except Exception:
        return 2, 16


@functools.lru_cache(maxsize=None)
def _make_transpose(vocab: int):
    n_tiles = (vocab + 127) // 128
    vocab_pad = n_tiles * 128
    K = 4
    n_strips = n_tiles // K
    tail_col = n_strips * K * 128
    tail_w = vocab - tail_col
    nc, ns = _sc_geometry()
    nw = nc * ns
    W_STRIP = K * 128

    mesh = plsc.VectorSubcoreMesh(core_axis_name="c", subcore_axis_name="s")

    @functools.partial(
        pl.kernel,
        mesh=mesh,
        out_type=jax.ShapeDtypeStruct((vocab_pad * ROW_W,), jnp.float32),
        scratch_types=[
            pltpu.VMEM((EMB_D, W_STRIP), jnp.float32),
            pltpu.VMEM((W_STRIP * ROW_W,), jnp.float32),
            pltpu.VMEM((tail_w * EMB_D,), jnp.float32),
        ],
        compiler_params=pltpu.CompilerParams(use_tc_tiling_on_sc=True,
                                             needs_layout_passes=False),
    )
    def transpose_kernel(tt_hbm, tail_hbm, out_hbm, in_v, out_v, tail_v):
        wid = lax.axis_index("s") * nc + lax.axis_index("c")
        lane = lax.broadcasted_iota(jnp.int32, (LANES,), 0)
        lane_dst = lane * ROW_W

        def do_strip(first_col):
            pltpu.sync_copy(tt_hbm.at[:, pl.ds(first_col, W_STRIP)], in_v)

            def per_group(g, carry):
                base = g * LANES * ROW_W
                col = g * LANES
                for d in range(EMB_D):
                    x = in_v[d, pl.ds(col, LANES)]
                    plsc.store_scatter(out_v, [lane_dst + (base + d)], x)
                return carry

            lax.fori_loop(0, W_STRIP // LANES, per_group, 0)
            pltpu.sync_copy(out_v,
                            out_hbm.at[pl.ds(first_col * ROW_W,
                                             W_STRIP * ROW_W)])

        n_mine = (n_strips + nw - 1) // nw

        def guarded(i, carry):
            t = i * nw + wid

            @pl.when(t < n_strips)
            def _():
                do_strip(t * W_STRIP)

            return carry

        lax.fori_loop(0, n_mine, guarded, 0)

        @pl.when(wid == 0)
        def _():
            pltpu.sync_copy(tail_hbm, tail_v)
            lane_src = lane * EMB_D
            for grp in range((tail_w + LANES - 1) // LANES):
                msk = (grp * LANES + lane) < tail_w
                for d in range(EMB_D):
                    x = plsc.load_gather(
                        tail_v, [lane_src + (grp * LANES * EMB_D + d)],
                        mask=msk)
                    plsc.store_scatter(
                        out_v, [lane_dst + (grp * LANES * ROW_W + d)],
                        x, mask=msk)
            pltpu.sync_copy(out_v.at[pl.ds(0, tail_w * ROW_W)],
                            out_hbm.at[pl.ds(tail_col * ROW_W,
                                             tail_w * ROW_W)])

    return transpose_kernel


@functools.lru_cache(maxsize=None)
def _make_gather(vocab_pad: int, batch: int, chunk: int):
    nc, ns = _sc_geometry()
    nw = nc * ns
    b_per_w = batch // nw
    n_pairs = b_per_w // (2 * chunk)
    assert b_per_w % (2 * chunk) == 0 and chunk % 8 == 0

    mesh = plsc.VectorSubcoreMesh(core_axis_name="c", subcore_axis_name="s")

    @functools.partial(
        pl.kernel,
        mesh=mesh,
        out_type=jax.ShapeDtypeStruct((batch, EMB_D), jnp.float32),
        scratch_types=[
            pltpu.VMEM((chunk,), jnp.int32),
            pltpu.VMEM((chunk,), jnp.int32),
            pltpu.VMEM((chunk, ROW_W), jnp.float32),
            pltpu.VMEM((chunk, ROW_W), jnp.float32),
            pltpu.SemaphoreType.DMA,
            pltpu.SemaphoreType.DMA,
            pltpu.SemaphoreType.DMA,
            pltpu.SemaphoreType.DMA,
        ],
        compiler_params=pltpu.CompilerParams(use_tc_tiling_on_sc=False),
    )
    def gather_kernel(table_hbm, idx_hbm, out_hbm, idx_a, idx_b, rows_a, rows_b,
                      sem_ga, sem_gb, sem_wa, sem_wb):
        wid = lax.axis_index("s") * nc + lax.axis_index("c")
        base = wid * b_per_w

        def pair(j, carry):
            off_a = base + (2 * j) * chunk
            off_b = off_a + chunk
            pltpu.sync_copy(idx_hbm.at[pl.ds(off_a, chunk)], idx_a)
            ga = pltpu.async_copy(table_hbm.at[idx_a], rows_a, sem_ga)
            pltpu.sync_copy(idx_hbm.at[pl.ds(off_b, chunk)], idx_b)
            gb = pltpu.async_copy(table_hbm.at[idx_b], rows_b, sem_gb)
            ga.wait()
            wa = pltpu.async_copy(rows_a.at[:, pl.ds(0, EMB_D)],
                                  out_hbm.at[pl.ds(off_a, chunk)], sem_wa)
            gb.wait()
            wb = pltpu.async_copy(rows_b.at[:, pl.ds(0, EMB_D)],
                                  out_hbm.at[pl.ds(off_b, chunk)], sem_wb)
            wa.wait()
            wb.wait()
            return carry

        lax.fori_loop(0, n_pairs, pair, 0)

    return gather_kernel


def kernel(input, weight):
    b, s = input.shape
    batch = b * s
    vocab = weight.shape[0]
    vocab_pad = ((vocab + 127) // 128) * 128
    idx = input.reshape(batch).astype(jnp.int32)
    tt = jnp.swapaxes(weight, 0, 1)
    tail_col = (vocab // (4 * 128)) * 4 * 128
    tail = jnp.reshape(weight[tail_col:], (-1,))
    flat = _make_transpose(vocab)(tt, tail)
    table = jnp.reshape(flat, (vocab_pad, ROW_W))
    out = _make_gather(vocab_pad, batch, 1280)(table, idx)
    return out.reshape(b, s, EMB_D)


# speedup vs baseline: 1.8862x; 1.8862x over previous
import functools

import jax
import jax.numpy as jnp
from jax import lax
from jax.experimental import pallas as pl
from jax.experimental.pallas import tpu as pltpu
from jax.experimental.pallas import tpu_sc as plsc

EMB_D = 32
LANES = 16
IN_W = 513


@functools.lru_cache(maxsize=None)
def _sc_geometry():
    try:
        info = plsc.get_sparse_core_info()
        return int(info.num_cores), int(info.num_subcores)
    except Exception:
        return 2, 16


@functools.lru_cache(maxsize=None)
def _make_transpose(vocab: int):
    n_tiles = (vocab + 127) // 128
    vocab_pad = n_tiles * 128
    K = 4
    n_strips = n_tiles // K
    tail_col = n_strips * K * 128
    tail_w = vocab - tail_col
    nc, ns = _sc_geometry()
    nw = nc * ns
    W_STRIP = K * 128
    n_pairs = (n_strips // nw + 1) // 2

    mesh = plsc.VectorSubcoreMesh(core_axis_name="c", subcore_axis_name="s")

    @functools.partial(
        pl.kernel,
        mesh=mesh,
        out_type=jax.ShapeDtypeStruct((vocab_pad * EMB_D,), jnp.float32),
        scratch_types=[
            pltpu.VMEM((EMB_D, IN_W), jnp.float32),
            pltpu.VMEM((EMB_D, IN_W), jnp.float32),
            pltpu.VMEM((W_STRIP * EMB_D,), jnp.float32),
            pltpu.VMEM((W_STRIP * EMB_D,), jnp.float32),
            pltpu.VMEM((tail_w * EMB_D,), jnp.float32),
            pltpu.SemaphoreType.DMA,
            pltpu.SemaphoreType.DMA,
            pltpu.SemaphoreType.DMA,
            pltpu.SemaphoreType.DMA,
        ],
        compiler_params=pltpu.CompilerParams(use_tc_tiling_on_sc=True,
                                             needs_layout_passes=False),
    )
    def transpose_kernel(tt_hbm, tail_hbm, out_hbm, in_a, in_b, out_a, out_b,
                         tail_v, sem_ia, sem_ib, sem_oa, sem_ob):
        wid = lax.axis_index("s") * nc + lax.axis_index("c")
        dlane = lax.broadcasted_iota(jnp.int32, (LANES,), 0)
        zeros = dlane * 0

        def stage(first_col, in_v, sem):
            return pltpu.async_copy(
                tt_hbm.at[:, pl.ds(first_col, W_STRIP)],
                in_v.at[:, pl.ds(0, W_STRIP)], sem)

        def transpose(in_v, out_v):
            def group(g, carry):
                for k in range(8):
                    v = g * 8 + k
                    vv = zeros + v
                    x0 = plsc.load_gather(in_v, [dlane, vv])
                    x1 = plsc.load_gather(in_v, [dlane + LANES, vv])
                    out_v[pl.ds(v * EMB_D, LANES)] = x0
                    out_v[pl.ds(v * EMB_D + LANES, LANES)] = x1
                return carry

            lax.fori_loop(0, W_STRIP // 8, group, 0)

        def unstage(first_col, out_v, sem):
            return pltpu.async_copy(
                out_v,
                out_hbm.at[pl.ds(first_col * EMB_D, W_STRIP * EMB_D)], sem)

        def pair(j, carry):
            ta = (2 * j) * nw + wid
            tb = ta + nw
            ca = stage(ta * W_STRIP, in_a, sem_ia)

            @pl.when(tb < n_strips)
            def _():
                stage(tb * W_STRIP, in_b, sem_ib)

            ca.wait()
            transpose(in_a, out_a)
            wa = unstage(ta * W_STRIP, out_a, sem_oa)

            @pl.when(tb < n_strips)
            def _():
                pltpu.make_async_copy(
                    tt_hbm.at[:, pl.ds(0, W_STRIP)],
                    in_b.at[:, pl.ds(0, W_STRIP)], sem_ib).wait()
                transpose(in_b, out_b)
                unstage(tb * W_STRIP, out_b, sem_ob).wait()

            wa.wait()
            return carry

        lax.fori_loop(0, n_pairs, pair, 0)

        @pl.when(wid == 0)
        def _():
            pltpu.sync_copy(tail_hbm, tail_v)
            pltpu.sync_copy(tail_v,
                            out_hbm.at[pl.ds(tail_col * EMB_D,
                                             tail_w * EMB_D)])

    return transpose_kernel


@functools.lru_cache(maxsize=None)
def _make_gather(vocab_pad: int, batch: int, chunk: int):
    nc, ns = _sc_geometry()
    nw = nc * ns
    b_per_w = batch // nw
    n_pairs = b_per_w // (2 * chunk)
    assert b_per_w % (2 * chunk) == 0 and chunk % 8 == 0

    mesh = plsc.VectorSubcoreMesh(core_axis_name="c", subcore_axis_name="s")

    @functools.partial(
        pl.kernel,
        mesh=mesh,
        out_type=jax.ShapeDtypeStruct((batch, EMB_D), jnp.float32),
        scratch_types=[
            pltpu.VMEM((chunk,), jnp.int32),
            pltpu.VMEM((chunk,), jnp.int32),
            pltpu.VMEM((chunk, EMB_D), jnp.float32),
            pltpu.VMEM((chunk, EMB_D), jnp.float32),
            pltpu.SemaphoreType.DMA,
            pltpu.SemaphoreType.DMA,
            pltpu.SemaphoreType.DMA,
            pltpu.SemaphoreType.DMA,
        ],
        compiler_params=pltpu.CompilerParams(use_tc_tiling_on_sc=False),
    )
    def gather_kernel(table_hbm, idx_hbm, out_hbm, idx_a, idx_b, rows_a, rows_b,
                      sem_ga, sem_gb, sem_wa, sem_wb):
        wid = lax.axis_index("s") * nc + lax.axis_index("c")
        base = wid * b_per_w

        def pair(j, carry):
            off_a = base + (2 * j) * chunk
            off_b = off_a + chunk
            pltpu.sync_copy(idx_hbm.at[pl.ds(off_a, chunk)], idx_a)
            ga = pltpu.async_copy(table_hbm.at[idx_a], rows_a, sem_ga)
            pltpu.sync_copy(idx_hbm.at[pl.ds(off_b, chunk)], idx_b)
            gb = pltpu.async_copy(table_hbm.at[idx_b], rows_b, sem_gb)
            ga.wait()
            wa = pltpu.async_copy(rows_a, out_hbm.at[pl.ds(off_a, chunk)], sem_wa)
            gb.wait()
            wb = pltpu.async_copy(rows_b, out_hbm.at[pl.ds(off_b, chunk)], sem_wb)
            wa.wait()
            wb.wait()
            return carry

        lax.fori_loop(0, n_pairs, pair, 0)

    return gather_kernel


def kernel(input, weight):
    b, s = input.shape
    batch = b * s
    vocab = weight.shape[0]
    vocab_pad = ((vocab + 127) // 128) * 128
    idx = input.reshape(batch).astype(jnp.int32)
    tt = jnp.swapaxes(weight, 0, 1)
    tail_col = (vocab // (4 * 128)) * 4 * 128
    tail = jnp.reshape(weight[tail_col:], (-1,))
    flat = _make_transpose(vocab)(tt, tail)
    table = jnp.reshape(flat, (vocab_pad, EMB_D))
    out = _make_gather(vocab_pad, batch, 1600)(table, idx)
    return out.reshape(b, s, EMB_D)


# speedup vs baseline: 2.5675x; 1.3612x over previous
import functools

import jax
import jax.numpy as jnp
from jax import lax
from jax.experimental import pallas as pl
from jax.experimental.pallas import tpu as pltpu
from jax.experimental.pallas import tpu_sc as plsc

EMB_D = 32
ROW_W = 40
LANES = 16


@functools.lru_cache(maxsize=None)
def _sc_geometry():
    try:
        info = plsc.get_sparse_core_info()
        return int(info.num_cores), int(info.num_subcores)
    except Exception:
        return 2, 16


@functools.lru_cache(maxsize=None)
def _make_transpose(vocab: int):
    n_tiles = (vocab + 127) // 128
    vocab_pad = n_tiles * 128
    K = 4
    n_strips = n_tiles // K
    tail_col = n_strips * K * 128
    tail_w = vocab - tail_col
    nc, ns = _sc_geometry()
    nw = nc * ns
    W_STRIP = K * 128
    n_pairs = (n_strips // nw + 1) // 2

    mesh = plsc.VectorSubcoreMesh(core_axis_name="c", subcore_axis_name="s")

    @functools.partial(
        pl.kernel,
        mesh=mesh,
        out_type=jax.ShapeDtypeStruct((vocab_pad * ROW_W,), jnp.float32),
        scratch_types=[
            pltpu.VMEM((EMB_D, W_STRIP), jnp.float32),
            pltpu.VMEM((EMB_D, W_STRIP), jnp.float32),
            pltpu.VMEM((W_STRIP * ROW_W,), jnp.float32),
            pltpu.VMEM((W_STRIP * ROW_W,), jnp.float32),
            pltpu.VMEM((tail_w * EMB_D,), jnp.float32),
            pltpu.SemaphoreType.DMA,
            pltpu.SemaphoreType.DMA,
            pltpu.SemaphoreType.DMA,
            pltpu.SemaphoreType.DMA,
        ],
        compiler_params=pltpu.CompilerParams(use_tc_tiling_on_sc=True,
                                             needs_layout_passes=False),
    )
    def transpose_kernel(tt_hbm, tail_hbm, out_hbm, in_a, in_b, out_a, out_b,
                         tail_v, sem_ia, sem_ib, sem_oa, sem_ob):
        wid = lax.axis_index("s") * nc + lax.axis_index("c")
        lane = lax.broadcasted_iota(jnp.int32, (LANES,), 0)
        lane_dst = lane * ROW_W

        def stage(first_col, in_v, sem):
            return pltpu.async_copy(
                tt_hbm.at[:, pl.ds(first_col, W_STRIP)], in_v, sem)

        def transpose(in_v, out_v):
            def group(g, carry):
                base = g * LANES * ROW_W
                col = g * LANES
                for d in range(EMB_D):
                    x = in_v[d, pl.ds(col, LANES)]
                    plsc.store_scatter(out_v, [lane_dst + (base + d)], x)
                return carry

            lax.fori_loop(0, W_STRIP // LANES, group, 0)

        def unstage(first_col, out_v, sem):
            return pltpu.async_copy(
                out_v,
                out_hbm.at[pl.ds(first_col * ROW_W, W_STRIP * ROW_W)], sem)

        def pair(j, carry):
            ta = (2 * j) * nw + wid
            tb = ta + nw
            ca = stage(ta * W_STRIP, in_a, sem_ia)

            @pl.when(tb < n_strips)
            def _():
                stage(tb * W_STRIP, in_b, sem_ib)

            ca.wait()
            transpose(in_a, out_a)
            wa = unstage(ta * W_STRIP, out_a, sem_oa)

            @pl.when(tb < n_strips)
            def _():
                pltpu.make_async_copy(
                    tt_hbm.at[:, pl.ds(0, W_STRIP)], in_b, sem_ib).wait()
                transpose(in_b, out_b)
                unstage(tb * W_STRIP, out_b, sem_ob).wait()

            wa.wait()
            return carry

        lax.fori_loop(0, n_pairs, pair, 0)

        @pl.when(wid == 0)
        def _():
            pltpu.sync_copy(tail_hbm, tail_v)
            lane_src = lane * EMB_D
            for grp in range((tail_w + LANES - 1) // LANES):
                msk = (grp * LANES + lane) < tail_w
                for d in range(EMB_D):
                    x = plsc.load_gather(
                        tail_v, [lane_src + (grp * LANES * EMB_D + d)],
                        mask=msk)
                    plsc.store_scatter(
                        out_a, [lane_dst + (grp * LANES * ROW_W + d)],
                        x, mask=msk)
            pltpu.sync_copy(out_a.at[pl.ds(0, tail_w * ROW_W)],
                            out_hbm.at[pl.ds(tail_col * ROW_W,
                                             tail_w * ROW_W)])

    return transpose_kernel


@functools.lru_cache(maxsize=None)
def _make_gather(vocab_pad: int, batch: int, chunk: int):
    nc, ns = _sc_geometry()
    nw = nc * ns
    b_per_w = batch // nw
    n_pairs = b_per_w // (2 * chunk)
    assert b_per_w % (2 * chunk) == 0 and chunk % 8 == 0

    mesh = plsc.VectorSubcoreMesh(core_axis_name="c", subcore_axis_name="s")

    @functools.partial(
        pl.kernel,
        mesh=mesh,
        out_type=jax.ShapeDtypeStruct((batch, EMB_D), jnp.float32),
        scratch_types=[
            pltpu.VMEM((chunk,), jnp.int32),
            pltpu.VMEM((chunk,), jnp.int32),
            pltpu.VMEM((chunk, ROW_W), jnp.float32),
            pltpu.VMEM((chunk, ROW_W), jnp.float32),
            pltpu.SemaphoreType.DMA,
            pltpu.SemaphoreType.DMA,
            pltpu.SemaphoreType.DMA,
            pltpu.SemaphoreType.DMA,
        ],
        compiler_params=pltpu.CompilerParams(use_tc_tiling_on_sc=False),
    )
    def gather_kernel(table_hbm, idx_hbm, out_hbm, idx_a, idx_b, rows_a, rows_b,
                      sem_ga, sem_gb, sem_wa, sem_wb):
        wid = lax.axis_index("s") * nc + lax.axis_index("c")
        base = wid * b_per_w

        def pair(j, carry):
            off_a = base + (2 * j) * chunk
            off_b = off_a + chunk
            pltpu.sync_copy(idx_hbm.at[pl.ds(off_a, chunk)], idx_a)
            ga = pltpu.async_copy(table_hbm.at[idx_a], rows_a, sem_ga)
            pltpu.sync_copy(idx_hbm.at[pl.ds(off_b, chunk)], idx_b)
            gb = pltpu.async_copy(table_hbm.at[idx_b], rows_b, sem_gb)
            ga.wait()
            wa = pltpu.async_copy(rows_a.at[:, pl.ds(0, EMB_D)],
                                  out_hbm.at[pl.ds(off_a, chunk)], sem_wa)
            gb.wait()
            wb = pltpu.async_copy(rows_b.at[:, pl.ds(0, EMB_D)],
                                  out_hbm.at[pl.ds(off_b, chunk)], sem_wb)
            wa.wait()
            wb.wait()
            return carry

        lax.fori_loop(0, n_pairs, pair, 0)

    return gather_kernel


def kernel(input, weight):
    b, s = input.shape
    batch = b * s
    vocab = weight.shape[0]
    vocab_pad = ((vocab + 127) // 128) * 128
    idx = input.reshape(batch).astype(jnp.int32)
    tt = jnp.swapaxes(weight, 0, 1)
    tail_col = (vocab // (4 * 128)) * 4 * 128
    tail = jnp.reshape(weight[tail_col:], (-1,))
    flat = _make_transpose(vocab)(tt, tail)
    table = jnp.reshape(flat, (vocab_pad, ROW_W))
    out = _make_gather(vocab_pad, batch, 1280)(table, idx)
    return out.reshape(b, s, EMB_D)
